# baseline (speedup 1.0000x reference)
import functools

import jax
import jax.numpy as jnp
from jax import lax
from jax.experimental import pallas as pl
from jax.experimental.pallas import tpu as pltpu
from jax.experimental.pallas import tpu_sc as plsc

BATCH = 16384
D = 16
VOC = 1_000_000
PACK = 128 // D
NBLK = VOC // PACK

_info = plsc.get_sparse_core_info()
NC = _info.num_cores
NS = _info.num_subcores
L = _info.num_lanes
NW = NC * NS
BPW = BATCH // NW
CHUNK = 128
NCHUNK = BPW // CHUNK
VECS = CHUNK // L

TCOLS = VOC // 128
STRIP_C = 8
STRIP_W = STRIP_C * 128
NSTRIP = TCOLS // STRIP_C
REMV = NSTRIP * STRIP_W
REMW = TCOLS * 128 - REMV
SPW = (NSTRIP + NW - 1) // NW

_mesh = plsc.VectorSubcoreMesh(core_axis_name="c", subcore_axis_name="s")


@functools.partial(
    pl.kernel,
    mesh=_mesh,
    out_type=(
        jax.ShapeDtypeStruct((NBLK, PACK * D), jnp.float32),
        jax.ShapeDtypeStruct((NBLK, PACK * D), jnp.float32),
    ),
    scratch_types=[
        pltpu.VMEM((2, D, STRIP_W + 1), jnp.float32),
        pltpu.VMEM((STRIP_W // PACK, PACK * D + 1), jnp.float32),
        pltpu.SemaphoreType.DMA,
    ],
    compiler_params=pltpu.CompilerParams(needs_layout_passes=False),
)
def _detile_sc(wut_hbm, wit_hbm, utail_hbm, itail_hbm, wu_out, wi_out,
               buf_v, out_v, sem):
    wid = lax.axis_index("s") * NC + lax.axis_index("c")
    lane = lax.iota(jnp.int32, L)

    def stage(src_hbm, s, b):
        v0 = pl.multiple_of(s * STRIP_W, 128)
        pltpu.async_copy(src_hbm.at[:, pl.ds(v0, STRIP_W)],
                         buf_v.at[b, :, pl.ds(0, STRIP_W)], sem)

    def transpose_store(dst_hbm, s, b, width):
        def row_body(q, _):
            for j in range(PACK):
                col = jnp.full((L,), q * PACK + j, jnp.int32)
                vals = plsc.load_gather(buf_v.at[b], [lane, col])
                out_v[q, pl.ds(j * D, D)] = vals
            return 0

        lax.fori_loop(0, width // PACK, row_body, 0)
        row0 = pl.multiple_of(s * (STRIP_W // PACK), 16)
        pltpu.sync_copy(out_v.at[pl.ds(0, width // PACK), pl.ds(0, PACK * D)],
                        dst_hbm.at[pl.ds(row0, width // PACK), :])

    def wait_strip(src_hbm, s, b):
        v0 = pl.multiple_of(s * STRIP_W, 128)
        pltpu.make_async_copy(src_hbm.at[:, pl.ds(v0, STRIP_W)],
                              buf_v.at[b, :, pl.ds(0, STRIP_W)], sem).wait()

    def run_table(src_hbm, dst_hbm):
        s0 = wid

        @pl.when(s0 < NSTRIP)
        def _():
            stage(src_hbm, s0, 0)
        s1 = wid + NW

        @pl.when(s1 < NSTRIP)
        def _():
            stage(src_hbm, s1, 1)

        def strip_pair(k, _):
            for b in range(2):
                s = wid + (2 * k + b) * NW

                @pl.when(s < NSTRIP)
                def _(s=s, b=b):
                    wait_strip(src_hbm, s, b)
                    nxt = s + 2 * NW

                    @pl.when(nxt < NSTRIP)
                    def _():
                        stage(src_hbm, nxt, b)
                    transpose_store(dst_hbm, s, b, STRIP_W)
            return 0

        lax.fori_loop(0, (SPW + 1) // 2, strip_pair, 0)

    run_table(wut_hbm, wu_out)
    run_table(wit_hbm, wi_out)

    @pl.when(wid == 0)
    def _():
        for src_hbm, dst_hbm in ((wut_hbm, wu_out), (wit_hbm, wi_out)):
            pltpu.sync_copy(src_hbm.at[:, pl.ds(REMV, REMW)],
                            buf_v.at[0, :, pl.ds(0, REMW)])

            def rem_body(q, _):
                for j in range(PACK):
                    col = jnp.full((L,), q * PACK + j, jnp.int32)
                    vals = plsc.load_gather(buf_v.at[0], [lane, col])
                    out_v[q, pl.ds(j * D, D)] = vals
                return 0

            lax.fori_loop(0, REMW // PACK, rem_body, 0)
            pltpu.sync_copy(
                out_v.at[pl.ds(0, REMW // PACK), pl.ds(0, PACK * D)],
                dst_hbm.at[pl.ds(REMV // PACK, REMW // PACK), :])

    @pl.when(wid == 1)
    def _():
        dst = out_v.at[pl.ds(0, PACK), pl.ds(0, PACK * D)]
        pltpu.sync_copy(utail_hbm, dst)
        pltpu.sync_copy(dst, wu_out.at[pl.ds(NBLK - PACK, PACK), :])

    @pl.when(wid == 2)
    def _():
        dst = out_v.at[pl.ds(PACK, PACK), pl.ds(0, PACK * D)]
        pltpu.sync_copy(itail_hbm, dst)
        pltpu.sync_copy(dst, wi_out.at[pl.ds(NBLK - PACK, PACK), :])


@functools.partial(
    pl.kernel,
    mesh=_mesh,
    out_type=jax.ShapeDtypeStruct((BATCH,), jnp.float32),
    scratch_types=[
        pltpu.VMEM((NCHUNK, CHUNK), jnp.int32),
        pltpu.VMEM((NCHUNK, CHUNK), jnp.int32),
        pltpu.VMEM((NCHUNK, CHUNK), jnp.int32),
        pltpu.VMEM((NCHUNK, CHUNK), jnp.int32),
        pltpu.VMEM((2, CHUNK, PACK * D), jnp.float32),
        pltpu.VMEM((2, CHUNK, PACK * D), jnp.float32),
        pltpu.VMEM((BPW,), jnp.float32),
        pltpu.SemaphoreType.DMA,
        pltpu.SemaphoreType.DMA,
    ],
    compiler_params=pltpu.CompilerParams(needs_layout_passes=False),
)
def _pmf_sc(uidx_hbm, iidx_hbm, wu_hbm, wi_hbm, out_hbm,
            uidx_v, iidx_v, ublk_v, iblk_v, urows_v, irows_v, out_v,
            usem, isem):
    wid = lax.axis_index("s") * NC + lax.axis_index("c")
    base_row = wid * NCHUNK

    pltpu.sync_copy(uidx_hbm.at[pl.ds(base_row, NCHUNK)], uidx_v)
    pltpu.sync_copy(iidx_hbm.at[pl.ds(base_row, NCHUNK)], iidx_v)

    for c in range(NCHUNK):
        for j in range(VECS):
            s = pl.ds(j * L, L)
            ublk_v[c, s] = lax.shift_right_logical(uidx_v[c, s], 3)
            iblk_v[c, s] = lax.shift_right_logical(iidx_v[c, s], 3)

    def start_chunk(c):
        b = c % 2
        ucp = pltpu.async_copy(wu_hbm.at[ublk_v.at[c]], urows_v.at[b], usem)
        icp = pltpu.async_copy(wi_hbm.at[iblk_v.at[c]], irows_v.at[b], isem)
        return ucp, icp

    lane = lax.iota(jnp.int32, L)
    inflight = start_chunk(0)

    for c in range(NCHUNK):
        ucp, icp = inflight
        ucp.wait()
        icp.wait()
        if c + 1 < NCHUNK:
            inflight = start_chunk(c + 1)
        b = c % 2
        ub = urows_v.at[b]
        ib = irows_v.at[b]

        def group_body(g, _, c=c, ub=ub, ib=ib):
            rows = g * L + lane
            s = pl.ds(g * L, L)
            ucol0 = (uidx_v[c, s] & (PACK - 1)) * D
            icol0 = (iidx_v[c, s] & (PACK - 1)) * D
            acc = jnp.zeros((L,), jnp.float32)
            for d in range(D):
                uc = plsc.load_gather(ub, [rows, ucol0 + d])
                ic = plsc.load_gather(ib, [rows, icol0 + d])
                acc = acc + uc * ic
            out_v[pl.ds(c * CHUNK + g * L, L)] = acc
            return 0

        lax.fori_loop(0, VECS, group_body, 0)

    pltpu.sync_copy(out_v, out_hbm.at[pl.ds(wid * BPW, BPW)])


def kernel(uesr_indices, item_indices, w_user, w_item):
    uidx = uesr_indices.astype(jnp.int32).reshape(NW * NCHUNK, CHUNK)
    iidx = item_indices.astype(jnp.int32).reshape(NW * NCHUNK, CHUNK)
    utail = w_user[TCOLS * 128:, :].reshape(PACK, PACK * D)
    itail = w_item[TCOLS * 128:, :].reshape(PACK, PACK * D)
    wu_lin, wi_lin = _detile_sc(w_user.T, w_item.T, utail, itail)
    return _pmf_sc(uidx, iidx, wu_lin, wi_lin)


# speedup vs baseline: 1.0633x; 1.0633x over previous
import functools

import jax
import jax.numpy as jnp
from jax import lax
from jax.experimental import pallas as pl
from jax.experimental.pallas import tpu as pltpu
from jax.experimental.pallas import tpu_sc as plsc

BATCH = 16384
D = 16

_info = plsc.get_sparse_core_info()
NC = _info.num_cores
NS = _info.num_subcores
L = _info.num_lanes
NW = NC * NS
BPW = BATCH // NW
CHUNK = 128
NCHUNK = BPW // CHUNK
GROUPS = BPW // L

_mesh = plsc.VectorSubcoreMesh(core_axis_name="c", subcore_axis_name="s")


@functools.partial(
    pl.kernel,
    mesh=_mesh,
    out_type=jax.ShapeDtypeStruct((BATCH,), jnp.float32),
    scratch_types=[
        pltpu.VMEM((NCHUNK, CHUNK), jnp.int32),
        pltpu.VMEM((NCHUNK, CHUNK), jnp.int32),
        pltpu.VMEM((BPW, D), jnp.float32),
        pltpu.VMEM((BPW, D), jnp.float32),
        pltpu.VMEM((BPW,), jnp.float32),
        pltpu.SemaphoreType.DMA,
        pltpu.SemaphoreType.DMA,
    ],
    compiler_params=pltpu.CompilerParams(
        needs_layout_passes=False, use_tc_tiling_on_sc=False
    ),
)
def _pmf_sc(uidx_hbm, iidx_hbm, wu_hbm, wi_hbm, out_hbm,
            uidx_v, iidx_v, urows_v, irows_v, out_v, usem, isem):
    wid = lax.axis_index("s") * NC + lax.axis_index("c")
    base_row = wid * NCHUNK

    pltpu.sync_copy(uidx_hbm.at[pl.ds(base_row, NCHUNK)], uidx_v)
    pltpu.sync_copy(iidx_hbm.at[pl.ds(base_row, NCHUNK)], iidx_v)

    ucopies = []
    icopies = []
    for c in range(NCHUNK):
        dst_u = urows_v.at[pl.ds(c * CHUNK, CHUNK), :]
        dst_i = irows_v.at[pl.ds(c * CHUNK, CHUNK), :]
        ucopies.append(pltpu.async_copy(wu_hbm.at[uidx_v.at[c]], dst_u, usem))
        icopies.append(pltpu.async_copy(wi_hbm.at[iidx_v.at[c]], dst_i, isem))
    for cp in ucopies:
        cp.wait()
    for cp in icopies:
        cp.wait()

    lane = lax.iota(jnp.int32, L)

    def group_body(g, _):
        row0 = g * L
        row_idx = row0 + lane
        acc = jnp.zeros((L,), jnp.float32)
        for d in range(D):
            col_idx = jnp.full((L,), d, jnp.int32)
            uc = plsc.load_gather(urows_v, [row_idx, col_idx])
            ic = plsc.load_gather(irows_v, [row_idx, col_idx])
            acc = acc + uc * ic
        out_v[pl.ds(row0, L)] = acc
        return 0

    lax.fori_loop(0, GROUPS, group_body, 0)

    pltpu.sync_copy(out_v, out_hbm.at[pl.ds(wid * BPW, BPW)])


def kernel(uesr_indices, item_indices, w_user, w_item):
    uidx = uesr_indices.astype(jnp.int32).reshape(NW * NCHUNK, CHUNK)
    iidx = item_indices.astype(jnp.int32).reshape(NW * NCHUNK, CHUNK)
    return _pmf_sc(uidx, iidx, w_user, w_item)
